# baseline (speedup 1.0000x reference)
import functools

import jax
import jax.numpy as jnp
import numpy as np
from jax.experimental import pallas as pl

_IN_C = 96
_OUT_C = 96
_KS = 3
_AREA = _KS * _KS
_CLUSTERS = 32
_MAX_ITER = 20
_TOL = 0.005


def _mlp3(x, W1, b1, W2, b2, W3, b3):
    h = jax.nn.relu(x @ W1 + b1)
    h = jax.nn.relu(h @ W2 + b2)
    return h @ W3 + b3


def _iter_kernel(m_ref, fsq_ref, csq_ref, f_ref, assign_ref, sum_ref, cnt_ref):
    t = pl.program_id(1)
    tpn = f_ref.shape[1]
    d2 = (fsq_ref[0] + csq_ref[0]) - 2.0 * m_ref[0]
    mn = jnp.min(d2, axis=1, keepdims=True)
    iota_k = jax.lax.broadcasted_iota(jnp.int32, (tpn, _CLUSTERS), 1)
    idx = jnp.min(jnp.where(d2 == mn, iota_k, _CLUSTERS), axis=1, keepdims=True)
    assign_ref[0] = idx
    onehot = (idx == iota_k).astype(jnp.float32)
    psum = jax.lax.dot_general(onehot, f_ref[0], (((0,), (0,)), ((), ())),
                               preferred_element_type=jnp.float32)
    pcnt = jnp.sum(onehot, axis=0, keepdims=True)

    @pl.when(t == 0)
    def _init():
        sum_ref[0] = psum
        cnt_ref[0] = pcnt

    @pl.when(t != 0)
    def _acc():
        sum_ref[0] += psum
        cnt_ref[0] += pcnt


def _kmeans_iter(m, fsq, csq, features):
    B, N, C = features.shape
    tpn = 1024
    while N % tpn:
        tpn //= 2
    assign, sums, counts = pl.pallas_call(
        _iter_kernel,
        grid=(B, N // tpn),
        in_specs=[
            pl.BlockSpec((1, tpn, _CLUSTERS), lambda b, t: (b, t, 0)),
            pl.BlockSpec((1, tpn, 1), lambda b, t: (b, t, 0)),
            pl.BlockSpec((1, 1, _CLUSTERS), lambda b, t: (b, 0, 0)),
            pl.BlockSpec((1, tpn, C), lambda b, t: (b, t, 0)),
        ],
        out_specs=[
            pl.BlockSpec((1, tpn, 1), lambda b, t: (b, t, 0)),
            pl.BlockSpec((1, _CLUSTERS, C), lambda b, t: (b, 0, 0)),
            pl.BlockSpec((1, 1, _CLUSTERS), lambda b, t: (b, 0, 0)),
        ],
        out_shape=[
            jax.ShapeDtypeStruct((B, N, 1), jnp.int32),
            jax.ShapeDtypeStruct((B, _CLUSTERS, C), jnp.float32),
            jax.ShapeDtypeStruct((B, 1, _CLUSTERS), jnp.float32),
        ],
    )(m, fsq, csq, features)
    return assign.reshape(B, N), sums, counts.reshape(B, _CLUSTERS)


def _kmeans(features, key):
    B, N, F = features.shape
    cent0 = jnp.stack([
        features[b][jax.random.permutation(jax.random.fold_in(key, b), N)[:_CLUSTERS]]
        for b in range(B)
    ])
    assign0 = jnp.zeros((B, N), dtype=jnp.int32)
    done0 = jnp.zeros((B,), dtype=bool)
    fsq = jnp.sum(features ** 2, axis=2, keepdims=True)

    def cond_fn(carry):
        i, cent, assign, done = carry
        return jnp.logical_and(i < _MAX_ITER, jnp.logical_not(jnp.all(done)))

    def body_fn(carry):
        i, cent, assign, done = carry
        m = jnp.stack([features[b] @ cent[b].T for b in range(B)])
        csq = jnp.sum(cent ** 2, axis=2)[:, None, :]
        new_assign, sums, counts = _kmeans_iter(m, fsq, csq, features)
        new_cent = jnp.where((counts > 0)[:, :, None], sums / jnp.maximum(counts, 1.0)[:, :, None], cent)
        shift = jnp.sum(jnp.linalg.norm(new_cent - cent, axis=2), axis=1)
        newly_done = shift < _TOL * N
        assign = jnp.where(done[:, None], assign, new_assign)
        cent = jnp.where(done[:, None, None], cent, new_cent)
        done = jnp.logical_or(done, newly_done)
        return (i + 1, cent, assign, done)

    _, _, assign, _ = jax.lax.while_loop(
        cond_fn, body_fn, (jnp.int32(0), cent0, assign0, done0)
    )
    return assign


def _rows_per_tile(H, Wp):
    best = 1
    for r in range(1, H + 1):
        if H % r == 0 and (r * Wp) % 128 == 0 and r * Wp <= 2048:
            best = r
    return best


def _patch_slabs(xu_ref, xc_ref, xd_ref, Wp, rows, tp):
    slabs = []
    for x_ref in (xu_ref, xc_ref, xd_ref):
        s = x_ref[0]
        left = jnp.concatenate([s[:, -1:], s[:, :-1]], axis=1)
        right = jnp.concatenate([s[:, 1:], s[:, :1]], axis=1)
        slabs += [left, s, right]
    return slabs


def _obs_kernel(xu_ref, xc_ref, xd_ref, o_ref, *, Wp, rows, tp):
    slabs = _patch_slabs(xu_ref, xc_ref, xd_ref, Wp, rows, tp)
    s = slabs[0]
    for a in range(1, _AREA):
        s = s + slabs[a]
    o_ref[0] = (s / jnp.float32(_AREA)).T


def _centers_kernel(idx_ref, xu_ref, xc_ref, xd_ref, sum_ref, cnt_ref, *, Wp, rows, tp):
    t = pl.program_id(1)
    slabs = _patch_slabs(xu_ref, xc_ref, xd_ref, Wp, rows, tp)
    p_t = jnp.concatenate(slabs, axis=0)
    onehot = (idx_ref[0] == jax.lax.broadcasted_iota(jnp.int32, (_CLUSTERS, tp), 0)).astype(jnp.float32)
    psum = jax.lax.dot_general(onehot, p_t, (((1,), (1,)), ((), ())),
                               preferred_element_type=jnp.float32)
    pcnt = jnp.sum(onehot, axis=1, keepdims=True)

    @pl.when(t == 0)
    def _init():
        sum_ref[0] = psum
        cnt_ref[0] = pcnt

    @pl.when(t != 0)
    def _acc():
        sum_ref[0] += psum
        cnt_ref[0] += pcnt


def _conv_kernel(idx_ref, xu_ref, xc_ref, xd_ref, w_ref, b_ref, o_ref, *, Wp, rows, tp):
    slabs = _patch_slabs(xu_ref, xc_ref, xd_ref, Wp, rows, tp)
    p_t = jnp.concatenate(slabs, axis=0).astype(jnp.bfloat16)
    idx = idx_ref[0]
    onehot = (idx == jax.lax.broadcasted_iota(jnp.int32, (_CLUSTERS, tp), 0)).astype(jnp.float32)
    bias_p = jnp.dot(b_ref[0], onehot, preferred_element_type=jnp.float32)
    acc = jnp.zeros((_OUT_C, tp), jnp.float32)
    for g in range(_CLUSTERS // 4):
        y = jnp.dot(w_ref[0, g], p_t, preferred_element_type=jnp.float32)
        for j in range(4):
            k = 4 * g + j
            acc = jnp.where(idx == k, y[j * _OUT_C:(j + 1) * _OUT_C], acc)
    o_ref[0] = acc + bias_p


def kernel(x, base_kernels, kg_W1, kg_b1, kg_W2, kg_b2, kg_W3, kg_b3, bg_W1, bg_b1, bg_W2, bg_b2, bg_W3, bg_b3):
    B, C, H, W = x.shape
    P = H * W
    F = C * _AREA
    Wp = ((W + 1 + 127) // 128) * 128
    rows = _rows_per_tile(H, Wp)
    tp = rows * Wp
    n_tiles = H // rows
    Pp = H * Wp
    grid = (B, n_tiles)

    xpad4 = jnp.pad(x, ((0, 0), (0, 0), (1, 1), (0, Wp - W)))
    x_sh = [xpad4[:, :, 1 + di:1 + di + H, :].reshape(B, C, Pp) for di in (-1, 0, 1)]
    x_spec = pl.BlockSpec((1, C, tp), lambda b, t: (b, 0, t))
    x_specs = [x_spec, x_spec, x_spec]

    obs_pad = pl.pallas_call(
        functools.partial(_obs_kernel, Wp=Wp, rows=rows, tp=tp),
        grid=grid,
        in_specs=x_specs,
        out_specs=pl.BlockSpec((1, tp, C), lambda b, t: (b, t, 0)),
        out_shape=jax.ShapeDtypeStruct((B, Pp, C), jnp.float32),
    )(*x_sh)
    obs = obs_pad.reshape(B, H, Wp, C)[:, :, :W, :].reshape(B, P, C)

    cluster_indices = _kmeans(obs, jax.random.key(42))

    idxp = jnp.pad(cluster_indices.reshape(B, H, W), ((0, 0), (0, 0), (0, Wp - W)),
                   constant_values=-1).reshape(B, 1, Pp)
    idx_spec = pl.BlockSpec((1, 1, tp), lambda b, t: (b, 0, t))

    sums, counts = pl.pallas_call(
        functools.partial(_centers_kernel, Wp=Wp, rows=rows, tp=tp),
        grid=grid,
        in_specs=[idx_spec] + x_specs,
        out_specs=[
            pl.BlockSpec((1, _CLUSTERS, F), lambda b, t: (b, 0, 0)),
            pl.BlockSpec((1, _CLUSTERS, 1), lambda b, t: (b, 0, 0)),
        ],
        out_shape=[
            jax.ShapeDtypeStruct((B, _CLUSTERS, F), jnp.float32),
            jax.ShapeDtypeStruct((B, _CLUSTERS, 1), jnp.float32),
        ],
    )(idxp, *x_sh)
    centers_am = sums / jnp.maximum(counts, 1.0)

    centers = centers_am.reshape(B, _CLUSTERS, _AREA, C).transpose(0, 1, 3, 2).reshape(B, _CLUSTERS, F)

    attn = jax.nn.softmax(_mlp3(centers, kg_W1, kg_b1, kg_W2, kg_b2, kg_W3, kg_b3), axis=-1)
    base_t = base_kernels.reshape(-1, C, _AREA, _OUT_C).transpose(0, 2, 1, 3).reshape(-1, F, _OUT_C)
    kernels_t = jnp.einsum('bkn,ncf->bkcf', attn, base_t.transpose(0, 2, 1)).astype(jnp.bfloat16)
    biases = _mlp3(centers, bg_W1, bg_b1, bg_W2, bg_b2, bg_W3, bg_b3)
    biases_t = biases.transpose(0, 2, 1)

    out_pad = pl.pallas_call(
        functools.partial(_conv_kernel, Wp=Wp, rows=rows, tp=tp),
        grid=grid,
        in_specs=[idx_spec] + x_specs + [
            pl.BlockSpec((1, _CLUSTERS // 4, 4 * _OUT_C, F), lambda b, t: (b, 0, 0, 0)),
            pl.BlockSpec((1, _OUT_C, _CLUSTERS), lambda b, t: (b, 0, 0)),
        ],
        out_specs=pl.BlockSpec((1, _OUT_C, tp), lambda b, t: (b, 0, t)),
        out_shape=jax.ShapeDtypeStruct((B, _OUT_C, Pp), jnp.float32),
    )(idxp, *x_sh, kernels_t.reshape(B, _CLUSTERS // 4, 4 * _OUT_C, F), biases_t)
    out = out_pad.reshape(B, _OUT_C, H, Wp)[:, :, :, :W]

    return (out, cluster_indices)


# speedup vs baseline: 1.0536x; 1.0536x over previous
import functools

import jax
import jax.numpy as jnp
import numpy as np
from jax.experimental import pallas as pl

_IN_C = 96
_OUT_C = 96
_KS = 3
_AREA = _KS * _KS
_CLUSTERS = 32
_MAX_ITER = 20
_TOL = 0.005


def _mlp3(x, W1, b1, W2, b2, W3, b3):
    h = jax.nn.relu(x @ W1 + b1)
    h = jax.nn.relu(h @ W2 + b2)
    return h @ W3 + b3


def _segsum_kernel(idx_ref, f_ref, sum_ref, cnt_ref):
    t = pl.program_id(1)
    tpn = f_ref.shape[1]
    onehot = (idx_ref[0] == jax.lax.broadcasted_iota(jnp.int32, (tpn, _CLUSTERS), 1)).astype(jnp.float32)
    psum = jax.lax.dot_general(onehot, f_ref[0], (((0,), (0,)), ((), ())),
                               preferred_element_type=jnp.float32)
    pcnt = jnp.sum(onehot, axis=0, keepdims=True)

    @pl.when(t == 0)
    def _init():
        sum_ref[0] = psum
        cnt_ref[0] = pcnt

    @pl.when(t != 0)
    def _acc():
        sum_ref[0] += psum
        cnt_ref[0] += pcnt


def _segment_mean_sums(features, assign):
    B, N, C = features.shape
    tpn = 1024
    while N % tpn:
        tpn //= 2
    sums, counts = pl.pallas_call(
        _segsum_kernel,
        grid=(B, N // tpn),
        in_specs=[
            pl.BlockSpec((1, tpn, 1), lambda b, t: (b, t, 0)),
            pl.BlockSpec((1, tpn, C), lambda b, t: (b, t, 0)),
        ],
        out_specs=[
            pl.BlockSpec((1, _CLUSTERS, C), lambda b, t: (b, 0, 0)),
            pl.BlockSpec((1, 1, _CLUSTERS), lambda b, t: (b, 0, 0)),
        ],
        out_shape=[
            jax.ShapeDtypeStruct((B, _CLUSTERS, C), jnp.float32),
            jax.ShapeDtypeStruct((B, 1, _CLUSTERS), jnp.float32),
        ],
    )(assign.reshape(B, N, 1), features)
    return sums, counts.reshape(B, _CLUSTERS)


def _kmeans(features, key):
    B, N, F = features.shape
    cent0 = jnp.stack([
        features[b][jax.random.permutation(jax.random.fold_in(key, b), N)[:_CLUSTERS]]
        for b in range(B)
    ])
    assign0 = jnp.zeros((B, N), dtype=jnp.int32)
    done0 = jnp.zeros((B,), dtype=bool)

    def cond_fn(carry):
        i, cent, assign, done = carry
        return jnp.logical_and(i < _MAX_ITER, jnp.logical_not(jnp.all(done)))

    def body_fn(carry):
        i, cent, assign, done = carry
        d2 = jnp.stack([
            jnp.sum(features[b] ** 2, axis=1, keepdims=True)
            + jnp.sum(cent[b] ** 2, axis=1)[None, :]
            - 2.0 * (features[b] @ cent[b].T)
            for b in range(B)
        ])
        new_assign = jnp.argmin(d2, axis=2).astype(jnp.int32)
        sums, counts = _segment_mean_sums(features, new_assign)
        new_cent = jnp.where((counts > 0)[:, :, None], sums / jnp.maximum(counts, 1.0)[:, :, None], cent)
        shift = jnp.sum(jnp.linalg.norm(new_cent - cent, axis=2), axis=1)
        newly_done = shift < _TOL * N
        assign = jnp.where(done[:, None], assign, new_assign)
        cent = jnp.where(done[:, None, None], cent, new_cent)
        done = jnp.logical_or(done, newly_done)
        return (i + 1, cent, assign, done)

    _, _, assign, _ = jax.lax.while_loop(
        cond_fn, body_fn, (jnp.int32(0), cent0, assign0, done0)
    )
    return assign


def _rows_per_tile(H, Wp):
    best = 1
    for r in range(1, H + 1):
        if H % r == 0 and (r * Wp) % 128 == 0 and r * Wp <= 2048:
            best = r
    return best


def _patch_slabs(xu_ref, xc_ref, xd_ref, Wp, rows, tp):
    slabs = []
    for x_ref in (xu_ref, xc_ref, xd_ref):
        s = x_ref[0]
        left = jnp.concatenate([s[:, -1:], s[:, :-1]], axis=1)
        right = jnp.concatenate([s[:, 1:], s[:, :1]], axis=1)
        slabs += [left, s, right]
    return slabs


def _obs_kernel(xu_ref, xc_ref, xd_ref, o_ref, *, Wp, rows, tp):
    slabs = _patch_slabs(xu_ref, xc_ref, xd_ref, Wp, rows, tp)
    s = slabs[0]
    for a in range(1, _AREA):
        s = s + slabs[a]
    o_ref[0] = (s / jnp.float32(_AREA)).T


def _centers_kernel(idx_ref, xu_ref, xc_ref, xd_ref, sum_ref, cnt_ref, *, Wp, rows, tp):
    t = pl.program_id(1)
    slabs = _patch_slabs(xu_ref, xc_ref, xd_ref, Wp, rows, tp)
    p_t = jnp.concatenate(slabs, axis=0)
    onehot = (idx_ref[0] == jax.lax.broadcasted_iota(jnp.int32, (_CLUSTERS, tp), 0)).astype(jnp.float32)
    psum = jax.lax.dot_general(onehot, p_t, (((1,), (1,)), ((), ())),
                               preferred_element_type=jnp.float32)
    pcnt = jnp.sum(onehot, axis=1, keepdims=True)

    @pl.when(t == 0)
    def _init():
        sum_ref[0] = psum
        cnt_ref[0] = pcnt

    @pl.when(t != 0)
    def _acc():
        sum_ref[0] += psum
        cnt_ref[0] += pcnt


def _conv_kernel(idx_ref, xu_ref, xc_ref, xd_ref, w_ref, b_ref, o_ref, *, Wp, rows, tp):
    slabs = _patch_slabs(xu_ref, xc_ref, xd_ref, Wp, rows, tp)
    p_t = jnp.concatenate(slabs, axis=0).astype(jnp.bfloat16)
    idx = idx_ref[0]
    onehot = (idx == jax.lax.broadcasted_iota(jnp.int32, (_CLUSTERS, tp), 0)).astype(jnp.float32)
    bias_p = jnp.dot(b_ref[0], onehot, preferred_element_type=jnp.float32)
    acc = jnp.zeros((_OUT_C, tp), jnp.float32)
    for g in range(_CLUSTERS // 4):
        y = jnp.dot(w_ref[0, g], p_t, preferred_element_type=jnp.float32)
        for j in range(4):
            k = 4 * g + j
            acc = jnp.where(idx == k, y[j * _OUT_C:(j + 1) * _OUT_C], acc)
    o_ref[0] = acc + bias_p


def kernel(x, base_kernels, kg_W1, kg_b1, kg_W2, kg_b2, kg_W3, kg_b3, bg_W1, bg_b1, bg_W2, bg_b2, bg_W3, bg_b3):
    B, C, H, W = x.shape
    P = H * W
    F = C * _AREA
    Wp = ((W + 1 + 127) // 128) * 128
    rows = _rows_per_tile(H, Wp)
    tp = rows * Wp
    n_tiles = H // rows
    Pp = H * Wp
    grid = (B, n_tiles)

    xpad4 = jnp.pad(x, ((0, 0), (0, 0), (1, 1), (0, Wp - W)))
    x_sh = [xpad4[:, :, 1 + di:1 + di + H, :].reshape(B, C, Pp) for di in (-1, 0, 1)]
    x_spec = pl.BlockSpec((1, C, tp), lambda b, t: (b, 0, t))
    x_specs = [x_spec, x_spec, x_spec]

    obs_pad = pl.pallas_call(
        functools.partial(_obs_kernel, Wp=Wp, rows=rows, tp=tp),
        grid=grid,
        in_specs=x_specs,
        out_specs=pl.BlockSpec((1, tp, C), lambda b, t: (b, t, 0)),
        out_shape=jax.ShapeDtypeStruct((B, Pp, C), jnp.float32),
    )(*x_sh)
    obs = obs_pad.reshape(B, H, Wp, C)[:, :, :W, :].reshape(B, P, C)

    cluster_indices = _kmeans(obs, jax.random.key(42))

    idxp = jnp.pad(cluster_indices.reshape(B, H, W), ((0, 0), (0, 0), (0, Wp - W)),
                   constant_values=-1).reshape(B, 1, Pp)
    idx_spec = pl.BlockSpec((1, 1, tp), lambda b, t: (b, 0, t))

    sums, counts = pl.pallas_call(
        functools.partial(_centers_kernel, Wp=Wp, rows=rows, tp=tp),
        grid=grid,
        in_specs=[idx_spec] + x_specs,
        out_specs=[
            pl.BlockSpec((1, _CLUSTERS, F), lambda b, t: (b, 0, 0)),
            pl.BlockSpec((1, _CLUSTERS, 1), lambda b, t: (b, 0, 0)),
        ],
        out_shape=[
            jax.ShapeDtypeStruct((B, _CLUSTERS, F), jnp.float32),
            jax.ShapeDtypeStruct((B, _CLUSTERS, 1), jnp.float32),
        ],
    )(idxp, *x_sh)
    centers_am = sums / jnp.maximum(counts, 1.0)

    centers = centers_am.reshape(B, _CLUSTERS, _AREA, C).transpose(0, 1, 3, 2).reshape(B, _CLUSTERS, F)

    attn = jax.nn.softmax(_mlp3(centers, kg_W1, kg_b1, kg_W2, kg_b2, kg_W3, kg_b3), axis=-1)
    base_t = base_kernels.reshape(-1, C, _AREA, _OUT_C).transpose(0, 2, 1, 3).reshape(-1, F, _OUT_C)
    kernels_t = jnp.einsum('bkn,ncf->bkcf', attn, base_t.transpose(0, 2, 1)).astype(jnp.bfloat16)
    biases = _mlp3(centers, bg_W1, bg_b1, bg_W2, bg_b2, bg_W3, bg_b3)
    biases_t = biases.transpose(0, 2, 1)

    rows_c = rows
    while rows_c * Wp > 1024 and rows_c % 2 == 0:
        rows_c //= 2
    tpc = rows_c * Wp
    xc_spec = pl.BlockSpec((1, C, tpc), lambda b, t: (b, 0, t))
    idxc_spec = pl.BlockSpec((1, 1, tpc), lambda b, t: (b, 0, t))
    out_pad = pl.pallas_call(
        functools.partial(_conv_kernel, Wp=Wp, rows=rows_c, tp=tpc),
        grid=(B, H // rows_c),
        in_specs=[idxc_spec, xc_spec, xc_spec, xc_spec] + [
            pl.BlockSpec((1, _CLUSTERS // 4, 4 * _OUT_C, F), lambda b, t: (b, 0, 0, 0)),
            pl.BlockSpec((1, _OUT_C, _CLUSTERS), lambda b, t: (b, 0, 0)),
        ],
        out_specs=pl.BlockSpec((1, _OUT_C, tpc), lambda b, t: (b, 0, t)),
        out_shape=jax.ShapeDtypeStruct((B, _OUT_C, Pp), jnp.float32),
    )(idxp, *x_sh, kernels_t.reshape(B, _CLUSTERS // 4, 4 * _OUT_C, F), biases_t)
    out = out_pad.reshape(B, _OUT_C, H, Wp)[:, :, :, :W]

    return (out, cluster_indices)


# speedup vs baseline: 1.0692x; 1.0148x over previous
import functools

import jax
import jax.numpy as jnp
import numpy as np
from jax.experimental import pallas as pl

_IN_C = 96
_OUT_C = 96
_KS = 3
_AREA = _KS * _KS
_CLUSTERS = 32
_MAX_ITER = 20
_TOL = 0.005


def _mlp3(x, W1, b1, W2, b2, W3, b3):
    h = jax.nn.relu(x @ W1 + b1)
    h = jax.nn.relu(h @ W2 + b2)
    return h @ W3 + b3


def _segsum_kernel(idx_ref, f_ref, sum_ref, cnt_ref):
    t = pl.program_id(1)
    tpn = f_ref.shape[1]
    onehot = (idx_ref[0] == jax.lax.broadcasted_iota(jnp.int32, (tpn, _CLUSTERS), 1)).astype(jnp.float32)
    psum = jax.lax.dot_general(onehot, f_ref[0], (((0,), (0,)), ((), ())),
                               preferred_element_type=jnp.float32)
    pcnt = jnp.sum(onehot, axis=0, keepdims=True)

    @pl.when(t == 0)
    def _init():
        sum_ref[0] = psum
        cnt_ref[0] = pcnt

    @pl.when(t != 0)
    def _acc():
        sum_ref[0] += psum
        cnt_ref[0] += pcnt


def _segment_mean_sums(features, assign):
    B, N, C = features.shape
    tpn = 1024
    while N % tpn:
        tpn //= 2
    sums, counts = pl.pallas_call(
        _segsum_kernel,
        grid=(B, N // tpn),
        in_specs=[
            pl.BlockSpec((1, tpn, 1), lambda b, t: (b, t, 0)),
            pl.BlockSpec((1, tpn, C), lambda b, t: (b, t, 0)),
        ],
        out_specs=[
            pl.BlockSpec((1, _CLUSTERS, C), lambda b, t: (b, 0, 0)),
            pl.BlockSpec((1, 1, _CLUSTERS), lambda b, t: (b, 0, 0)),
        ],
        out_shape=[
            jax.ShapeDtypeStruct((B, _CLUSTERS, C), jnp.float32),
            jax.ShapeDtypeStruct((B, 1, _CLUSTERS), jnp.float32),
        ],
    )(assign.reshape(B, N, 1), features)
    return sums, counts.reshape(B, _CLUSTERS)


def _kmeans(features, key):
    B, N, F = features.shape
    cent0 = jnp.stack([
        features[b][jax.random.permutation(jax.random.fold_in(key, b), N)[:_CLUSTERS]]
        for b in range(B)
    ])
    assign0 = jnp.zeros((B, N), dtype=jnp.int32)
    done0 = jnp.zeros((B,), dtype=bool)

    def cond_fn(carry):
        i, cent, assign, done = carry
        return jnp.logical_and(i < _MAX_ITER, jnp.logical_not(jnp.all(done)))

    def body_fn(carry):
        i, cent, assign, done = carry
        d2 = jnp.stack([
            jnp.sum(features[b] ** 2, axis=1, keepdims=True)
            + jnp.sum(cent[b] ** 2, axis=1)[None, :]
            - 2.0 * (features[b] @ cent[b].T)
            for b in range(B)
        ])
        new_assign = jnp.argmin(d2, axis=2).astype(jnp.int32)
        sums, counts = _segment_mean_sums(features, new_assign)
        new_cent = jnp.where((counts > 0)[:, :, None], sums / jnp.maximum(counts, 1.0)[:, :, None], cent)
        shift = jnp.sum(jnp.linalg.norm(new_cent - cent, axis=2), axis=1)
        newly_done = shift < _TOL * N
        assign = jnp.where(done[:, None], assign, new_assign)
        cent = jnp.where(done[:, None, None], cent, new_cent)
        done = jnp.logical_or(done, newly_done)
        return (i + 1, cent, assign, done)

    _, _, assign, _ = jax.lax.while_loop(
        cond_fn, body_fn, (jnp.int32(0), cent0, assign0, done0)
    )
    return assign


def _rows_per_tile(H, Wp):
    best = 1
    for r in range(1, H + 1):
        if H % r == 0 and (r * Wp) % 128 == 0 and r * Wp <= 2048:
            best = r
    return best


def _patch_slabs(xu_ref, xc_ref, xd_ref, Wp, rows, tp):
    slabs = []
    for x_ref in (xu_ref, xc_ref, xd_ref):
        s = x_ref[0]
        left = jnp.concatenate([s[:, -1:], s[:, :-1]], axis=1)
        right = jnp.concatenate([s[:, 1:], s[:, :1]], axis=1)
        slabs += [left, s, right]
    return slabs


def _obs_kernel(xu_ref, xc_ref, xd_ref, o_ref, *, Wp, rows, tp):
    slabs = _patch_slabs(xu_ref, xc_ref, xd_ref, Wp, rows, tp)
    s = slabs[0]
    for a in range(1, _AREA):
        s = s + slabs[a]
    o_ref[0] = (s / jnp.float32(_AREA)).T


def _centers_kernel(idx_ref, xu_ref, xc_ref, xd_ref, sum_ref, cnt_ref, *, Wp, rows, tp):
    t = pl.program_id(1)
    slabs = _patch_slabs(xu_ref, xc_ref, xd_ref, Wp, rows, tp)
    p_t = jnp.concatenate(slabs, axis=0)
    onehot = (idx_ref[0] == jax.lax.broadcasted_iota(jnp.int32, (_CLUSTERS, tp), 0)).astype(jnp.float32)
    psum = jax.lax.dot_general(onehot, p_t, (((1,), (1,)), ((), ())),
                               preferred_element_type=jnp.float32)
    pcnt = jnp.sum(onehot, axis=1, keepdims=True)

    @pl.when(t == 0)
    def _init():
        sum_ref[0] = psum
        cnt_ref[0] = pcnt

    @pl.when(t != 0)
    def _acc():
        sum_ref[0] += psum
        cnt_ref[0] += pcnt


def _conv_kernel(idx_ref, xu_ref, xc_ref, xd_ref, w_ref, b_ref, o_ref, *, Wp, rows, tp):
    slabs = _patch_slabs(xu_ref, xc_ref, xd_ref, Wp, rows, tp)
    p_t = jnp.concatenate(slabs, axis=0).astype(jnp.bfloat16)
    idx = idx_ref[0]
    onehot = (idx == jax.lax.broadcasted_iota(jnp.int32, (_CLUSTERS, tp), 0)).astype(jnp.float32)
    bias_p = jnp.dot(b_ref[0], onehot, preferred_element_type=jnp.float32)
    nh = max(1, tp // 1024)
    hw = tp // nh
    outs = []
    for h in range(nh):
        ph = p_t[:, h * hw:(h + 1) * hw]
        idx_h = idx[:, h * hw:(h + 1) * hw]
        acc = jnp.zeros((_OUT_C, hw), jnp.float32)
        for g in range(_CLUSTERS // 4):
            y = jnp.dot(w_ref[0, g], ph, preferred_element_type=jnp.float32)
            for j in range(4):
                k = 4 * g + j
                acc = jnp.where(idx_h == k, y[j * _OUT_C:(j + 1) * _OUT_C], acc)
        outs.append(acc)
    o_ref[0] = jnp.concatenate(outs, axis=1) + bias_p


def kernel(x, base_kernels, kg_W1, kg_b1, kg_W2, kg_b2, kg_W3, kg_b3, bg_W1, bg_b1, bg_W2, bg_b2, bg_W3, bg_b3):
    B, C, H, W = x.shape
    P = H * W
    F = C * _AREA
    Wp = ((W + 1 + 127) // 128) * 128
    rows = _rows_per_tile(H, Wp)
    tp = rows * Wp
    n_tiles = H // rows
    Pp = H * Wp
    grid = (B, n_tiles)

    xpad4 = jnp.pad(x, ((0, 0), (0, 0), (1, 1), (0, Wp - W)))
    x_sh = [xpad4[:, :, 1 + di:1 + di + H, :].reshape(B, C, Pp) for di in (-1, 0, 1)]
    x_spec = pl.BlockSpec((1, C, tp), lambda b, t: (b, 0, t))
    x_specs = [x_spec, x_spec, x_spec]

    obs_pad = pl.pallas_call(
        functools.partial(_obs_kernel, Wp=Wp, rows=rows, tp=tp),
        grid=grid,
        in_specs=x_specs,
        out_specs=pl.BlockSpec((1, tp, C), lambda b, t: (b, t, 0)),
        out_shape=jax.ShapeDtypeStruct((B, Pp, C), jnp.float32),
    )(*x_sh)
    obs = obs_pad.reshape(B, H, Wp, C)[:, :, :W, :].reshape(B, P, C)

    cluster_indices = _kmeans(obs, jax.random.key(42))

    idxp = jnp.pad(cluster_indices.reshape(B, H, W), ((0, 0), (0, 0), (0, Wp - W)),
                   constant_values=-1).reshape(B, 1, Pp)
    idx_spec = pl.BlockSpec((1, 1, tp), lambda b, t: (b, 0, t))

    sums, counts = pl.pallas_call(
        functools.partial(_centers_kernel, Wp=Wp, rows=rows, tp=tp),
        grid=grid,
        in_specs=[idx_spec] + x_specs,
        out_specs=[
            pl.BlockSpec((1, _CLUSTERS, F), lambda b, t: (b, 0, 0)),
            pl.BlockSpec((1, _CLUSTERS, 1), lambda b, t: (b, 0, 0)),
        ],
        out_shape=[
            jax.ShapeDtypeStruct((B, _CLUSTERS, F), jnp.float32),
            jax.ShapeDtypeStruct((B, _CLUSTERS, 1), jnp.float32),
        ],
    )(idxp, *x_sh)
    centers_am = sums / jnp.maximum(counts, 1.0)

    centers = centers_am.reshape(B, _CLUSTERS, _AREA, C).transpose(0, 1, 3, 2).reshape(B, _CLUSTERS, F)

    attn = jax.nn.softmax(_mlp3(centers, kg_W1, kg_b1, kg_W2, kg_b2, kg_W3, kg_b3), axis=-1)
    base_t = base_kernels.reshape(-1, C, _AREA, _OUT_C).transpose(0, 2, 1, 3).reshape(-1, F, _OUT_C)
    kernels_t = jnp.einsum('bkn,ncf->bkcf', attn, base_t.transpose(0, 2, 1)).astype(jnp.bfloat16)
    biases = _mlp3(centers, bg_W1, bg_b1, bg_W2, bg_b2, bg_W3, bg_b3)
    biases_t = biases.transpose(0, 2, 1)

    rows_c = rows
    tpc = rows_c * Wp
    xc_spec = pl.BlockSpec((1, C, tpc), lambda b, t: (b, 0, t))
    idxc_spec = pl.BlockSpec((1, 1, tpc), lambda b, t: (b, 0, t))
    out_pad = pl.pallas_call(
        functools.partial(_conv_kernel, Wp=Wp, rows=rows_c, tp=tpc),
        grid=(B, H // rows_c),
        in_specs=[idxc_spec, xc_spec, xc_spec, xc_spec] + [
            pl.BlockSpec((1, _CLUSTERS // 4, 4 * _OUT_C, F), lambda b, t: (b, 0, 0, 0)),
            pl.BlockSpec((1, _OUT_C, _CLUSTERS), lambda b, t: (b, 0, 0)),
        ],
        out_specs=pl.BlockSpec((1, _OUT_C, tpc), lambda b, t: (b, 0, t)),
        out_shape=jax.ShapeDtypeStruct((B, _OUT_C, Pp), jnp.float32),
    )(idxp, *x_sh, kernels_t.reshape(B, _CLUSTERS // 4, 4 * _OUT_C, F), biases_t)
    out = out_pad.reshape(B, _OUT_C, H, Wp)[:, :, :, :W]

    return (out, cluster_indices)
